# Initial kernel scaffold; baseline (speedup 1.0000x reference)
import jax
import jax.numpy as jnp
from jax.experimental import pallas as pl


def kernel(x, edge_index, edge_type, edge_weight, W0_0, b0_0, g0_0, be0_0, W0_1, b0_1, g0_1, be0_1, W1_0, b1_0, g1_0, be1_0, W1_1, b1_1, g1_1, be1_1, pW1, pb1, pW2, pb2):
    raise NotImplementedError("write your pallas kernel here")


# speedup vs baseline: 9.1944x; 9.1944x over previous
import functools

import jax
import jax.numpy as jnp
from jax import lax
from jax.experimental import pallas as pl
from jax.experimental.pallas import tpu as pltpu
from jax.experimental.pallas import tpu_sc as plsc

N = 10000
NP = 10240
NP2 = 2 * NP
E = 320000
D = 128
H = 256
F = 64
NCH = 4
NC = 2
NS = 16
W = 512
NWIN = E // W
ROWS = E // 128
KMAX = -(-NWIN // NS)
TS = NP2 // NS

BLK = 512
NB = NP // BLK

@functools.cache
def _sc_mesh():
    return plsc.VectorSubcoreMesh(
        core_axis_name="c", subcore_axis_name="s",
        num_cores=NC, num_subcores=NS)


def _rsqrt_newton(x):
    xi = lax.bitcast_convert_type(x, jnp.int32)
    yi = jnp.int32(0x5F3759DF) - lax.shift_right_logical(xi, 1)
    y = lax.bitcast_convert_type(yi, jnp.float32)
    for _ in range(3):
        y = y * (1.5 - 0.5 * x * y * y)
    return y


@functools.cache
def _sc_prep_kernel():
    return pl.kernel(
        _sc_prep_body,
        out_type=(
            jax.ShapeDtypeStruct((ROWS, 128), jnp.int32),
            jax.ShapeDtypeStruct((ROWS, 128), jnp.int32),
            jax.ShapeDtypeStruct((NP2,), jnp.float32),
        ),
        mesh=_sc_mesh(),
        scratch_types=[
            pltpu.VMEM((4, 128), jnp.int32),
            pltpu.VMEM((4, 128), jnp.int32),
            pltpu.VMEM((4, 128), jnp.int32),
            pltpu.VMEM((4, 128), jnp.float32),
            pltpu.VMEM((TS,), jnp.float32),
            pltpu.VMEM_SHARED((NP2,), jnp.float32),
        ],
    )


def _sc_prep_body(row2d, col2d, typ2d, ew2d, rowp, colp, dinv,
                  av, tv, ov, wv, dv, deg_sh):
    cid = lax.axis_index("c")
    sid = lax.axis_index("s")

    @pl.when(cid == 0)
    def _deg_dinv():
        def zero(i, _):
            dv[pl.ds(16 * i, 16)] = jnp.zeros((16,), jnp.float32)
            return 0
        lax.fori_loop(0, TS // 16, zero, 0)
        pltpu.sync_copy(dv, deg_sh.at[pl.ds(sid * TS, TS)])
        plsc.subcore_barrier()

        def win(k, _):
            widx = sid + NS * k

            @pl.when(widx < NWIN)
            def _():
                b4 = 4 * widx
                pltpu.sync_copy(col2d.at[pl.ds(b4, 4)], av)
                pltpu.sync_copy(typ2d.at[pl.ds(b4, 4)], tv)
                pltpu.sync_copy(ew2d.at[pl.ds(b4, 4)], wv)
                for g in range(4):
                    for i in range(8):
                        s = pl.ds(16 * i, 16)
                        ov[g, s] = av[g, s] + tv[g, s] * NP
                for g in range(4):
                    pltpu.sync_copy(wv.at[g], deg_sh.at[ov.at[g]], add=True)
            return 0
        lax.fori_loop(0, KMAX, win, 0)
        plsc.subcore_barrier()

        pltpu.sync_copy(deg_sh.at[pl.ds(sid * TS, TS)], dv)

        def inv(i, _):
            s = pl.ds(16 * i, 16)
            dv[s] = _rsqrt_newton(dv[s] + 1.0)
            return 0
        lax.fori_loop(0, TS // 16, inv, 0)
        pltpu.sync_copy(dv, dinv.at[pl.ds(sid * TS, TS)])

    @pl.when(cid == 1)
    def _indices():
        def win(k, _):
            widx = sid + NS * k

            @pl.when(widx < NWIN)
            def _():
                b4 = 4 * widx
                pltpu.sync_copy(typ2d.at[pl.ds(b4, 4)], tv)
                pltpu.sync_copy(row2d.at[pl.ds(b4, 4)], av)
                for g in range(4):
                    for i in range(8):
                        s = pl.ds(16 * i, 16)
                        ov[g, s] = av[g, s] + tv[g, s] * NP
                pltpu.sync_copy(ov, rowp.at[pl.ds(b4, 4)])
                pltpu.sync_copy(col2d.at[pl.ds(b4, 4)], av)
                for g in range(4):
                    for i in range(8):
                        s = pl.ds(16 * i, 16)
                        ov[g, s] = av[g, s] + tv[g, s] * NP
                pltpu.sync_copy(ov, colp.at[pl.ds(b4, 4)])
            return 0
        lax.fori_loop(0, KMAX, win, 0)


@functools.cache
def _sc_agg_kernel():
    return pl.kernel(
        _sc_agg_body,
        out_type=jax.ShapeDtypeStruct((NCH * NP2, F), jnp.float32),
        mesh=_sc_mesh(),
        compiler_params=pltpu.CompilerParams(use_tc_tiling_on_sc=False),
        scratch_types=[
            pltpu.VMEM((4, 128), jnp.int32),
            pltpu.VMEM((4, 128), jnp.int32),
            pltpu.VMEM((W,), jnp.float32),
            pltpu.VMEM((W, F), jnp.float32),
            pltpu.VMEM((128, F), jnp.float32),
            pltpu.VMEM_SHARED((NP2, F), jnp.float32),
        ],
    )


def _sc_agg_body(xwc, rowp, colp, ew1d, out,
                 rowv, colv, ewv, gbuf, zbuf, acc_sh):
    cid = lax.axis_index("c")
    sid = lax.axis_index("s")

    def zzero(i, _):
        zbuf[i >> 2, pl.ds(16 * (i & 3), 16)] = jnp.zeros((16,), jnp.float32)
        return 0
    lax.fori_loop(0, 128 * 4, zzero, 0)

    for ci in range(2):
        chunk = 2 * cid + ci
        cbase = chunk * NP2
        for z in range(10):
            pltpu.sync_copy(zbuf, acc_sh.at[pl.ds(sid * TS + 128 * z, 128)])
        plsc.subcore_barrier()

        def win(k, _):
            widx = sid + NS * k

            @pl.when(widx < NWIN)
            def _():
                b4 = 4 * widx
                pltpu.sync_copy(rowp.at[pl.ds(b4, 4)], rowv)
                pltpu.sync_copy(colp.at[pl.ds(b4, 4)], colv)
                pltpu.sync_copy(ew1d.at[pl.ds(W * widx, W)], ewv)
                for g in range(4):
                    for i in range(8):
                        s = pl.ds(16 * i, 16)
                        rowv[g, s] = rowv[g, s] + cbase
                for g in range(4):
                    pltpu.sync_copy(xwc.at[rowv.at[g]],
                                    gbuf.at[pl.ds(128 * g, 128)])

                @plsc.parallel_loop(0, W // 16, 1, unroll=2)
                def _mul(gi):
                    wg = ewv[pl.ds(16 * gi, 16)]
                    for i in range(16):
                        e = 16 * gi + i
                        sv = jnp.full((16,), wg[i], jnp.float32)
                        for f in range(4):
                            s = pl.ds(16 * f, 16)
                            gbuf[e, s] = gbuf[e, s] * sv

                for g in range(4):
                    pltpu.sync_copy(gbuf.at[pl.ds(128 * g, 128)],
                                    acc_sh.at[colv.at[g]], add=True)
            return 0
        lax.fori_loop(0, KMAX, win, 0)
        plsc.subcore_barrier()

        for z in range(5):
            off = sid * TS + 256 * z
            pltpu.sync_copy(acc_sh.at[pl.ds(off, 256)],
                            out.at[pl.ds(cbase + off, 256)])


def _mm0_body(x_ref, dinv_ref, w_ref, out_ref):
    xs = x_ref[...] * dinv_ref[...]
    out_ref[0] = jnp.dot(xs, w_ref[0, 0], preferred_element_type=jnp.float32)


def _tc_mm0(xpad, dinv2, W0s):
    return pl.pallas_call(
        _mm0_body,
        grid=(2, NCH, NB),
        in_specs=[
            pl.BlockSpec((BLK, D), lambda t, j, nb: (nb, 0)),
            pl.BlockSpec((BLK, 1), lambda t, j, nb: (t * NB + nb, 0)),
            pl.BlockSpec((1, 1, D, F), lambda t, j, nb: (t, j, 0, 0)),
        ],
        out_specs=pl.BlockSpec((1, BLK, F), lambda t, j, nb: (j, t * NB + nb, 0)),
        out_shape=jax.ShapeDtypeStruct((NCH, NP2, F), jnp.float32),
    )(xpad, dinv2, W0s)


def _stats_body(acc_ref, xwc_ref, dinv_ref, t1_ref, st_ref):
    nb = pl.program_id(2)
    v = dinv_ref[...] * (acc_ref[0] + xwc_ref[0])
    t1_ref[0] = v

    @pl.when(nb == 0)
    def _():
        st_ref[...] = jnp.zeros_like(st_ref)

    st_ref[0, 0, 0, :] += jnp.sum(v, axis=0)
    st_ref[0, 0, 1, :] += jnp.sum(v * v, axis=0)


def _tc_stats(acc, xwc, dinv2):
    return pl.pallas_call(
        _stats_body,
        grid=(2, NCH, NB),
        in_specs=[
            pl.BlockSpec((1, BLK, F), lambda t, j, nb: (j, t * NB + nb, 0)),
            pl.BlockSpec((1, BLK, F), lambda t, j, nb: (j, t * NB + nb, 0)),
            pl.BlockSpec((BLK, 1), lambda t, j, nb: (t * NB + nb, 0)),
        ],
        out_specs=[
            pl.BlockSpec((1, BLK, F), lambda t, j, nb: (j, t * NB + nb, 0)),
            pl.BlockSpec((1, 1, 2, F), lambda t, j, nb: (t, j, 0, 0)),
        ],
        out_shape=[
            jax.ShapeDtypeStruct((NCH, NP2, F), jnp.float32),
            jax.ShapeDtypeStruct((2, NCH, 2, F), jnp.float32),
        ],
    )(acc, xwc, dinv2)


def _bnmm_body(t1_ref, st_ref, gbe_ref, dinv_ref, w_ref, out_ref):
    nb = pl.program_id(2)
    dv = dinv_ref[...]
    z = jnp.zeros((BLK, F), jnp.float32)
    for jk in range(NCH):
        mu = st_ref[0, jk, 0, :] * (1.0 / N)
        var = st_ref[0, jk, 1, :] * (1.0 / N) - mu * mu
        inv = lax.rsqrt(var + 1e-5)
        g = gbe_ref[0, 0, 64 * jk:64 * jk + 64]
        be = gbe_ref[0, 1, 64 * jk:64 * jk + 64]
        h = (t1_ref[jk] - mu) * (inv * g) + be
        h = jnp.maximum(h, 0.0) * dv
        z += jnp.dot(h, w_ref[0, 0, 64 * jk:64 * jk + 64, :],
                     preferred_element_type=jnp.float32)
    pos = lax.broadcasted_iota(jnp.int32, (BLK, F), 0) + nb * BLK
    out_ref[0] = jnp.where(pos < N, z, 0.0)


def _tc_bnmm(t1, st, gbe, dinv2, Wl):
    return pl.pallas_call(
        _bnmm_body,
        grid=(2, NCH, NB),
        in_specs=[
            pl.BlockSpec((NCH, BLK, F), lambda t, j, nb: (0, t * NB + nb, 0)),
            pl.BlockSpec((1, NCH, 2, F), lambda t, j, nb: (t, 0, 0, 0)),
            pl.BlockSpec((1, 2, H), lambda t, j, nb: (t, 0, 0)),
            pl.BlockSpec((BLK, 1), lambda t, j, nb: (t * NB + nb, 0)),
            pl.BlockSpec((1, 1, H, F), lambda t, j, nb: (t, j, 0, 0)),
        ],
        out_specs=pl.BlockSpec((1, BLK, F), lambda t, j, nb: (j, t * NB + nb, 0)),
        out_shape=jax.ShapeDtypeStruct((NCH, NP2, F), jnp.float32),
    )(t1, st, gbe, dinv2, Wl)


def _head_body(t2a_ref, t2b_ref, st_ref, gbe_ref, pw1_ref, pb1_ref,
               pw2_ref, pb2_ref, out_ref):
    z = jnp.zeros((BLK, H), jnp.float32) + pb1_ref[...]
    for t in range(2):
        tb = t2a_ref if t == 0 else t2b_ref
        for jk in range(NCH):
            mu = st_ref[t, jk, 0, :] * (1.0 / N)
            var = st_ref[t, jk, 1, :] * (1.0 / N) - mu * mu
            inv = lax.rsqrt(var + 1e-5)
            g = gbe_ref[t, 0, 64 * jk:64 * jk + 64]
            be = gbe_ref[t, 1, 64 * jk:64 * jk + 64]
            h = (tb[jk] - mu) * (inv * g) + be
            h = jnp.maximum(h, 0.0)
            z += jnp.dot(h, pw1_ref[t * H + 64 * jk:t * H + 64 * jk + 64, :],
                         preferred_element_type=jnp.float32)
    z = jnp.maximum(z, 0.0)
    out_ref[...] = jnp.dot(z, pw2_ref[...],
                           preferred_element_type=jnp.float32) + pb2_ref[...]


def _tc_head(t2, st, gbe, pW1, pb1, pW2, pb2):
    return pl.pallas_call(
        _head_body,
        grid=(NB,),
        in_specs=[
            pl.BlockSpec((NCH, BLK, F), lambda nb: (0, nb, 0)),
            pl.BlockSpec((NCH, BLK, F), lambda nb: (0, NB + nb, 0)),
            pl.BlockSpec((2, NCH, 2, F), lambda nb: (0, 0, 0, 0)),
            pl.BlockSpec((2, 2, H), lambda nb: (0, 0, 0)),
            pl.BlockSpec((2 * H, H), lambda nb: (0, 0)),
            pl.BlockSpec((H,), lambda nb: (0,)),
            pl.BlockSpec((H, 1), lambda nb: (0, 0)),
            pl.BlockSpec((1,), lambda nb: (0,)),
        ],
        out_specs=pl.BlockSpec((BLK, 1), lambda nb: (nb, 0)),
        out_shape=jax.ShapeDtypeStruct((NP, 1), jnp.float32),
    )(t2, t2, st, gbe, pW1, pb1, pW2, pb2)


def kernel(x, edge_index, edge_type, edge_weight,
           W0_0, b0_0, g0_0, be0_0, W0_1, b0_1, g0_1, be0_1,
           W1_0, b1_0, g1_0, be1_0, W1_1, b1_1, g1_1, be1_1,
           pW1, pb1, pW2, pb2):
    del b0_0, b0_1, b1_0, b1_1

    row2d = edge_index[0].reshape(ROWS, 128)
    col2d = edge_index[1].reshape(ROWS, 128)
    typ2d = edge_type.reshape(ROWS, 128)
    ew2d = edge_weight.reshape(ROWS, 128)

    rowp, colp, dinv = _sc_prep_kernel()(row2d, col2d, typ2d, ew2d)
    dinv2 = dinv.reshape(NP2, 1)

    xpad = jnp.pad(x, ((0, NP - N), (0, 0)))
    W0s = jnp.stack([W0_0, W1_0]).reshape(2, D, NCH, F).transpose(0, 2, 1, 3)
    W1s = jnp.stack([W0_1, W1_1]).reshape(2, H, NCH, F).transpose(0, 2, 1, 3)
    gbe0 = jnp.stack([jnp.stack([g0_0, be0_0]), jnp.stack([g1_0, be1_0])])
    gbe1 = jnp.stack([jnp.stack([g0_1, be0_1]), jnp.stack([g1_1, be1_1])])

    xwc0 = _tc_mm0(xpad, dinv2, W0s)
    acc0 = _sc_agg_kernel()(xwc0.reshape(NCH * NP2, F), rowp, colp, edge_weight)
    t1, st0 = _tc_stats(acc0.reshape(NCH, NP2, F), xwc0, dinv2)

    xwc1 = _tc_bnmm(t1, st0, gbe0, dinv2, W1s)
    acc1 = _sc_agg_kernel()(xwc1.reshape(NCH * NP2, F), rowp, colp, edge_weight)
    t2, st1 = _tc_stats(acc1.reshape(NCH, NP2, F), xwc1, dinv2)

    out = _tc_head(t2, st1, gbe1, pW1, pb1, pW2, pb2)
    return out[:N]
